# baseline (speedup 1.0000x reference)
import jax
import jax.numpy as jnp
from jax import lax
from jax.experimental import pallas as pl
from jax.experimental.pallas import tpu as pltpu
from jax.experimental.pallas import tpu_sc as plsc

N = 100000
D = 128
LANES = 16
NC = 2
NS = 16
NW = NC * NS

GCH = 128
CH = 256
NB = 3
NCHUNK = 12
BIG = 3128
SMALL = 3120
NBIG = 20
TAIL_BIG = BIG - NCHUNK * CH
TAIL_SMALL = SMALL - NCHUNK * CH


def _body(ordering_hbm, noise_hbm, table_hbm, mean_hbm, lstd_hbm, out_hbm,
          idx_all, buf_v, tbuf_v, mean_v, lstd_v,
          gsem0, gsem1, gsem2, nsem0, nsem1, nsem2,
          osem0, osem1, osem2, xsem, msem, tsem, tosem):
    gsem = [gsem0, gsem1, gsem2]
    nsem = [nsem0, nsem1, nsem2]
    osem = [osem0, osem1, osem2]

    wid = lax.axis_index("s") * NC + lax.axis_index("c")
    is_big = wid < NBIG
    base8 = 390 * wid + jnp.minimum(wid, NBIG)
    base_w = 8 * base8
    tail_start = base_w + NCHUNK * CH

    def chunk_start(c):
        return 8 * (base8 + (CH // 8) * c)

    pltpu.async_copy(ordering_hbm.at[pl.ds(base_w, SMALL)],
                     idx_all.at[pl.ds(0, SMALL)], xsem)

    @pl.when(is_big)
    def _():
        pltpu.async_copy(ordering_hbm.at[pl.ds(tail_start + TAIL_SMALL, 8)],
                         idx_all.at[pl.ds(SMALL, 8)], xsem)

    for c0 in range(NB):
        pltpu.async_copy(noise_hbm.at[pl.ds(chunk_start(c0), CH)],
                         buf_v.at[c0], nsem[c0])
    pltpu.async_copy(mean_hbm, mean_v, msem)
    pltpu.async_copy(lstd_hbm, lstd_v, msem)

    @pl.when(is_big)
    def _():
        pltpu.async_copy(noise_hbm.at[pl.ds(tail_start, TAIL_BIG)],
                         tbuf_v.at[pl.ds(0, TAIL_BIG)], tsem)

    @pl.when(jnp.logical_not(is_big))
    def _():
        pltpu.async_copy(noise_hbm.at[pl.ds(tail_start, TAIL_SMALL)],
                         tbuf_v.at[pl.ds(0, TAIL_SMALL)], tsem)

    pltpu.make_async_copy(mean_hbm, mean_v, msem).wait()
    pltpu.make_async_copy(lstd_hbm, lstd_v, msem).wait()
    mean_vals = [mean_v[pl.ds(j * LANES, LANES)] for j in range(D // LANES)]
    scale_vals = [jnp.exp(lstd_v[pl.ds(j * LANES, LANES)])
                  for j in range(D // LANES)]

    pltpu.make_async_copy(ordering_hbm.at[pl.ds(0, SMALL)],
                          idx_all.at[pl.ds(0, SMALL)], xsem).wait()

    @pl.when(is_big)
    def _():
        pltpu.make_async_copy(ordering_hbm.at[pl.ds(0, 8)],
                              idx_all.at[pl.ds(SMALL, 8)], xsem).wait()

    def issue_gadd(c, buf):
        pltpu.async_copy(table_hbm.at[idx_all.at[pl.ds(c * CH, GCH)]],
                         buf_v.at[buf, pl.ds(0, GCH)], gsem[buf], add=True)
        pltpu.async_copy(table_hbm.at[idx_all.at[pl.ds(c * CH + GCH, GCH)]],
                         buf_v.at[buf, pl.ds(GCH, GCH)], gsem[buf], add=True)

    def wait_gadd(buf):
        pltpu.make_async_copy(table_hbm.at[idx_all.at[pl.ds(0, GCH)]],
                              buf_v.at[buf, pl.ds(0, GCH)], gsem[buf]).wait()
        pltpu.make_async_copy(table_hbm.at[idx_all.at[pl.ds(0, GCH)]],
                              buf_v.at[buf, pl.ds(GCH, GCH)], gsem[buf]).wait()

    def wait_noise(buf):
        pltpu.make_async_copy(noise_hbm.at[pl.ds(0, CH)], buf_v.at[buf],
                              nsem[buf]).wait()

    def wait_out(buf):
        pltpu.make_async_copy(buf_v.at[buf], out_hbm.at[pl.ds(0, CH)],
                              osem[buf]).wait()

    def compute_scale(buf, nrows):
        def row_body(r, _):
            for j in range(D // LANES):
                sl = pl.ds(j * LANES, LANES)
                buf_v[buf, r, sl] = (mean_vals[j]
                                     + scale_vals[j] * buf_v[buf, r, sl])
            return None
        lax.fori_loop(0, nrows, row_body, None)

    for c in range(NCHUNK):
        b = c % NB
        bp = (c - 1) % NB
        bn = (c + 1) % NB
        wait_noise(b)
        compute_scale(b, CH)
        issue_gadd(c, b)
        if c >= 1:
            wait_gadd(bp)
            pltpu.async_copy(buf_v.at[bp],
                             out_hbm.at[pl.ds(chunk_start(c - 1), CH)],
                             osem[bp])
        if NB <= c + 1 < NCHUNK:
            wait_out(bn)
            pltpu.async_copy(noise_hbm.at[pl.ds(chunk_start(c + 1), CH)],
                             buf_v.at[bn], nsem[bn])

    wait_gadd((NCHUNK - 1) % NB)
    pltpu.async_copy(buf_v.at[(NCHUNK - 1) % NB],
                     out_hbm.at[pl.ds(chunk_start(NCHUNK - 1), CH)],
                     osem[(NCHUNK - 1) % NB])

    def tail(nrows):
        pltpu.make_async_copy(noise_hbm.at[pl.ds(0, nrows)],
                              tbuf_v.at[pl.ds(0, nrows)], tsem).wait()

        def row_body(r, _):
            for j in range(D // LANES):
                sl = pl.ds(j * LANES, LANES)
                tbuf_v[r, sl] = mean_vals[j] + scale_vals[j] * tbuf_v[r, sl]
            return None
        lax.fori_loop(0, nrows, row_body, None)
        toff = NCHUNK * CH
        pltpu.async_copy(table_hbm.at[idx_all.at[pl.ds(toff, nrows)]],
                         tbuf_v.at[pl.ds(0, nrows)], tsem, add=True)
        pltpu.make_async_copy(table_hbm.at[idx_all.at[pl.ds(0, nrows)]],
                              tbuf_v.at[pl.ds(0, nrows)], tsem).wait()
        pltpu.async_copy(tbuf_v.at[pl.ds(0, nrows)],
                         out_hbm.at[pl.ds(tail_start, nrows)], tosem)
        pltpu.make_async_copy(tbuf_v.at[pl.ds(0, nrows)],
                              out_hbm.at[pl.ds(0, nrows)], tosem).wait()

    @pl.when(is_big)
    def _():
        tail(TAIL_BIG)

    @pl.when(jnp.logical_not(is_big))
    def _():
        tail(TAIL_SMALL)

    wait_out(0)
    wait_out(1)
    wait_out(2)


@jax.jit
def _run(ordering, noise, pos_embed, rand_mean, rand_log_std):
    mesh = plsc.VectorSubcoreMesh(core_axis_name="c", subcore_axis_name="s",
                                  num_cores=NC, num_subcores=NS)
    f = pl.kernel(
        _body,
        out_type=jax.ShapeDtypeStruct((N, D), jnp.float32),
        mesh=mesh,
        scratch_types=[
            pltpu.VMEM((BIG,), jnp.int32),
            pltpu.VMEM((NB, CH, D), jnp.float32),
            pltpu.VMEM((TAIL_BIG, D), jnp.float32),
            pltpu.VMEM((D,), jnp.float32),
            pltpu.VMEM((D,), jnp.float32),
        ] + [pltpu.SemaphoreType.DMA] * 13,
    )
    return f(ordering, noise, pos_embed, rand_mean, rand_log_std)


def kernel(ordering, noise, pos_embed, rand_mean, rand_log_std):
    return _run(ordering.astype(jnp.int32), noise, pos_embed,
                rand_mean, rand_log_std)


# speedup vs baseline: 1.1393x; 1.1393x over previous
import jax
import jax.numpy as jnp
from jax import lax
from jax.experimental import pallas as pl
from jax.experimental.pallas import tpu as pltpu
from jax.experimental.pallas import tpu_sc as plsc

N = 100000
D = 128
LANES = 16
NC = 2
NS = 16
NW = NC * NS

CH = 128
NB = 3
NT = 8
NCHUNK = NB * NT
BIG = 3128
SMALL = 3120
NBIG = 20
TAIL_BIG = BIG - NCHUNK * CH
TAIL_SMALL = SMALL - NCHUNK * CH


def _body(ordering_hbm, noise_hbm, table_hbm, mean_hbm, lstd_hbm, out_hbm,
          idx_all, idx0_v, rows_v, noise_v, trows_v, tnoise_v, mean_v, lstd_v,
          gsem0, gsem1, gsem2, nsem0, nsem1, nsem2,
          osem0, osem1, osem2, xsem, x0sem, msem, tsem, tosem):
    gsem = [gsem0, gsem1, gsem2]
    nsem = [nsem0, nsem1, nsem2]
    osem = [osem0, osem1, osem2]

    wid = lax.axis_index("s") * NC + lax.axis_index("c")
    is_big = wid < NBIG
    base8 = 390 * wid + jnp.minimum(wid, NBIG)
    base_w = 8 * base8
    tail_start = base_w + NCHUNK * CH

    def chunk_start(c):
        return 8 * (base8 + (CH // 8) * c)

    pltpu.async_copy(ordering_hbm.at[pl.ds(base_w, CH)], idx0_v, x0sem)
    pltpu.async_copy(ordering_hbm.at[pl.ds(base_w, SMALL)],
                     idx_all.at[pl.ds(0, SMALL)], xsem)

    @pl.when(is_big)
    def _():
        pltpu.async_copy(ordering_hbm.at[pl.ds(tail_start + TAIL_SMALL, 8)],
                         idx_all.at[pl.ds(SMALL, 8)], xsem)

    pltpu.async_copy(noise_hbm.at[pl.ds(chunk_start(0), CH)],
                     noise_v.at[0], nsem[0])
    pltpu.async_copy(noise_hbm.at[pl.ds(chunk_start(1), CH)],
                     noise_v.at[1], nsem[1])
    pltpu.async_copy(mean_hbm, mean_v, msem)
    pltpu.async_copy(lstd_hbm, lstd_v, msem)

    @pl.when(is_big)
    def _():
        pltpu.async_copy(noise_hbm.at[pl.ds(tail_start, TAIL_BIG)],
                         tnoise_v.at[pl.ds(0, TAIL_BIG)], tsem)

    @pl.when(jnp.logical_not(is_big))
    def _():
        pltpu.async_copy(noise_hbm.at[pl.ds(tail_start, TAIL_SMALL)],
                         tnoise_v.at[pl.ds(0, TAIL_SMALL)], tsem)

    pltpu.make_async_copy(ordering_hbm.at[pl.ds(0, CH)], idx0_v,
                          x0sem).wait()
    pltpu.async_copy(table_hbm.at[idx0_v], rows_v.at[0], gsem[0])
    pltpu.make_async_copy(ordering_hbm.at[pl.ds(0, SMALL)],
                          idx_all.at[pl.ds(0, SMALL)], xsem).wait()

    @pl.when(is_big)
    def _():
        pltpu.make_async_copy(ordering_hbm.at[pl.ds(0, 8)],
                              idx_all.at[pl.ds(SMALL, 8)], xsem).wait()

    pltpu.async_copy(table_hbm.at[idx_all.at[pl.ds(CH, CH)]], rows_v.at[1],
                     gsem[1])
    TOFF = NCHUNK * CH

    @pl.when(is_big)
    def _():
        pltpu.async_copy(table_hbm.at[idx_all.at[pl.ds(TOFF, TAIL_BIG)]],
                         trows_v.at[pl.ds(0, TAIL_BIG)], tsem)

    @pl.when(jnp.logical_not(is_big))
    def _():
        pltpu.async_copy(table_hbm.at[idx_all.at[pl.ds(TOFF, TAIL_SMALL)]],
                         trows_v.at[pl.ds(0, TAIL_SMALL)], tsem)

    pltpu.make_async_copy(mean_hbm, mean_v, msem).wait()
    pltpu.make_async_copy(lstd_hbm, lstd_v, msem).wait()
    mean_vals = [mean_v[pl.ds(j * LANES, LANES)] for j in range(D // LANES)]
    scale_vals = [jnp.exp(lstd_v[pl.ds(j * LANES, LANES)])
                  for j in range(D // LANES)]

    def issue_fetch(c, buf):
        pltpu.async_copy(
            table_hbm.at[idx_all.at[pl.ds(c * CH, CH)]], rows_v.at[buf],
            gsem[buf])
        pltpu.async_copy(noise_hbm.at[pl.ds(chunk_start(c), CH)],
                         noise_v.at[buf], nsem[buf])

    def wait_fetch(buf):
        pltpu.make_async_copy(table_hbm.at[idx_all.at[pl.ds(0, CH)]],
                              rows_v.at[buf], gsem[buf]).wait()
        pltpu.make_async_copy(noise_hbm.at[pl.ds(0, CH)], noise_v.at[buf],
                              nsem[buf]).wait()

    def wait_out(buf):
        pltpu.make_async_copy(rows_v.at[buf], out_hbm.at[pl.ds(0, CH)],
                              osem[buf]).wait()

    def compute_rows(buf, nrows):
        def row_body(h, _):
            r = 2 * h
            for dr in range(2):
                for j in range(D // LANES):
                    sl = pl.ds(j * LANES, LANES)
                    rows_v[buf, r + dr, sl] = (
                        rows_v[buf, r + dr, sl] + mean_vals[j]
                        + scale_vals[j] * noise_v[buf, r + dr, sl])
            return None
        lax.fori_loop(0, nrows // 2, row_body, None)

    @pl.loop(0, NT)
    def _triple(t):
        c0 = NB * t
        for i in range(NB):
            b = i
            b2 = (i + 2) % NB
            c = c0 + i
            wait_fetch(b)
            if i == 0:
                @pl.when(t > 0)
                def _():
                    wait_out(b2)
                issue_fetch(c + 2, b2)
            else:
                @pl.when(t < NT - 1)
                def _():
                    wait_out(b2)
                    issue_fetch(c + 2, b2)
            compute_rows(b, CH)
            pltpu.async_copy(rows_v.at[b],
                             out_hbm.at[pl.ds(chunk_start(c), CH)], osem[b])

    def tail(nrows):
        pltpu.make_async_copy(table_hbm.at[idx_all.at[pl.ds(0, nrows)]],
                              trows_v.at[pl.ds(0, nrows)], tsem).wait()
        pltpu.make_async_copy(noise_hbm.at[pl.ds(0, nrows)],
                              tnoise_v.at[pl.ds(0, nrows)], tsem).wait()

        def row_body(r, _):
            for j in range(D // LANES):
                sl = pl.ds(j * LANES, LANES)
                trows_v[r, sl] = (trows_v[r, sl] + mean_vals[j]
                                  + scale_vals[j] * tnoise_v[r, sl])
            return None
        lax.fori_loop(0, nrows, row_body, None)
        pltpu.async_copy(trows_v.at[pl.ds(0, nrows)],
                         out_hbm.at[pl.ds(tail_start, nrows)], tosem)
        pltpu.make_async_copy(trows_v.at[pl.ds(0, nrows)],
                              out_hbm.at[pl.ds(0, nrows)], tosem).wait()

    @pl.when(is_big)
    def _():
        tail(TAIL_BIG)

    @pl.when(jnp.logical_not(is_big))
    def _():
        tail(TAIL_SMALL)

    wait_out(0)
    wait_out(1)
    wait_out(2)


@jax.jit
def _run(ordering, noise, pos_embed, rand_mean, rand_log_std):
    mesh = plsc.VectorSubcoreMesh(core_axis_name="c", subcore_axis_name="s",
                                  num_cores=NC, num_subcores=NS)
    f = pl.kernel(
        _body,
        out_type=jax.ShapeDtypeStruct((N, D), jnp.float32),
        mesh=mesh,
        scratch_types=[
            pltpu.VMEM((BIG,), jnp.int32),
            pltpu.VMEM((CH,), jnp.int32),
            pltpu.VMEM((NB, CH, D), jnp.float32),
            pltpu.VMEM((NB, CH, D), jnp.float32),
            pltpu.VMEM((TAIL_BIG, D), jnp.float32),
            pltpu.VMEM((TAIL_BIG, D), jnp.float32),
            pltpu.VMEM((D,), jnp.float32),
            pltpu.VMEM((D,), jnp.float32),
        ] + [pltpu.SemaphoreType.DMA] * 14,
    )
    return f(ordering, noise, pos_embed, rand_mean, rand_log_std)


def kernel(ordering, noise, pos_embed, rand_mean, rand_log_std):
    return _run(ordering.astype(jnp.int32), noise, pos_embed,
                rand_mean, rand_log_std)
